# baseline (speedup 1.0000x reference)
import functools

import jax
import jax.numpy as jnp
from jax import lax
from jax.experimental import pallas as pl
from jax.experimental.pallas import tpu as pltpu
from jax.experimental.pallas import tpu_sc as plsc

N = 10000
E = 320000
K = 4
IN = 32
OUT = 32
F = K * IN

NC = 2
NS = 16
NW = NC * NS

EPT = E // NW
CHUNK = 80
NCHUNK = EPT // CHUNK

HROWS = 640
ROWS_PER_TILE = N // NS

_MESH = plsc.VectorSubcoreMesh(core_axis_name="c", subcore_axis_name="s")


@functools.partial(
    pl.kernel,
    out_type=jax.ShapeDtypeStruct((NW, 2, HROWS * 16), jnp.float32),
    mesh=_MESH,
    scratch_types=[
        pltpu.VMEM((EPT,), jnp.int32),
        pltpu.VMEM((HROWS * 16,), jnp.float32),
        pltpu.VMEM((HROWS * 16,), jnp.float32),
    ],
    compiler_params=pltpu.CompilerParams(needs_layout_passes=False, use_tc_tiling_on_sc=False),
)
def _sc_degrees(src_hbm, dst_hbm, out_hbm, idx_v, hist_s, hist_d):
    c = lax.axis_index("c")
    s = lax.axis_index("s")
    wid = c * NS + s
    zero16 = jnp.zeros((16,), jnp.float32)
    ones16 = jnp.ones((16,), jnp.float32)

    @pl.loop(0, HROWS, unroll=8)
    def _zero(r):
        hist_s[pl.ds(r * 16, 16)] = zero16
        hist_d[pl.ds(r * 16, 16)] = zero16

    base_e = wid * EPT
    pltpu.sync_copy(src_hbm.at[pl.ds(base_e, EPT)], idx_v)

    @pl.loop(0, EPT // 16, unroll=8)
    def _hs(i):
        v = idx_v[pl.ds(i * 16, 16)]
        plsc.addupdate_scatter(hist_s, [v], ones16)

    pltpu.sync_copy(dst_hbm.at[pl.ds(base_e, EPT)], idx_v)

    @pl.loop(0, EPT // 16, unroll=8)
    def _hd(i):
        v = idx_v[pl.ds(i * 16, 16)]
        plsc.addupdate_scatter(hist_d, [v], ones16)

    pltpu.sync_copy(hist_s, out_hbm.at[wid, 0])
    pltpu.sync_copy(hist_d, out_hbm.at[wid, 1])


NFULL = EPT // 128
TAIL = EPT - NFULL * 128


@functools.partial(
    pl.kernel,
    out_type=jax.ShapeDtypeStruct((NC, N, F), jnp.float32),
    mesh=_MESH,
    scratch_types=[
        pltpu.VMEM((EPT,), jnp.int32),
        pltpu.VMEM((128,), jnp.int32),
        pltpu.VMEM((128,), jnp.int32),
        pltpu.VMEM((TAIL,), jnp.int32),
        pltpu.VMEM((128, F), jnp.float32),
        pltpu.VMEM((128, F), jnp.float32),
        pltpu.VMEM((25, F), jnp.float32),
        pltpu.VMEM_SHARED((N, F), jnp.float32),
        pltpu.SemaphoreType.DMA,
        pltpu.SemaphoreType.DMA,
        pltpu.SemaphoreType.DMA,
        pltpu.SemaphoreType.DMA,
        pltpu.SemaphoreType.DMA,
        pltpu.SemaphoreType.DMA,
    ],
    compiler_params=pltpu.CompilerParams(needs_layout_passes=False, use_tc_tiling_on_sc=False),
)
def _sc_aggregate(g_hbm, src_hbm, dst_hbm, out_hbm,
                  sidx_v, dwin0_v, dwin1_v, dtail_v, rows0_v, rows1_v,
                  zbuf_v, acc, gsem0, gsem1, dsem0, dsem1, ssem0, ssem1):
    c = lax.axis_index("c")
    s = lax.axis_index("s")
    wid = c * NS + s
    zero16 = jnp.zeros((16,), jnp.float32)

    @pl.loop(0, 25)
    def _zr(r):
        @pl.loop(0, F // 16)
        def _zc(q):
            zbuf_v[r, pl.ds(q * 16, 16)] = zero16

    row0 = s * ROWS_PER_TILE

    @pl.loop(0, ROWS_PER_TILE // 25)
    def _za(b):
        pltpu.sync_copy(zbuf_v, acc.at[pl.ds(row0 + b * 25, 25)])

    base_e = wid * EPT
    pltpu.sync_copy(src_hbm.at[pl.ds(base_e, EPT)], sidx_v)

    plsc.subcore_barrier()

    rows = (rows0_v, rows1_v)
    dwin = (dwin0_v, dwin1_v)
    gsems = (gsem0, gsem1)
    dsems = (dsem0, dsem1)
    ssems = (ssem0, ssem1)

    def _issue(cc, b):
        off = pl.multiple_of(cc * 128, 8)
        pltpu.async_copy(g_hbm.at[sidx_v.at[pl.ds(off, 128)]], rows[b],
                         gsems[b])
        pltpu.async_copy(dst_hbm.at[pl.ds(base_e + off, 128)], dwin[b],
                         dsems[b])

    def _scatter_issue(b):
        pltpu.make_async_copy(g_hbm.at[pl.ds(0, 128)], rows[b],
                              gsems[b]).wait()
        pltpu.make_async_copy(dst_hbm.at[pl.ds(0, 128)], dwin[b],
                              dsems[b]).wait()
        pltpu.async_copy(rows[b], acc.at[dwin[b]], ssems[b], add=True)

    def _scatter_wait(b):
        pltpu.make_async_copy(rows[b], acc.at[dwin[b]], ssems[b]).wait()

    _issue(0, 0)
    _issue(1, 1)

    @pl.loop(0, NFULL // 2)
    def _chunks(j):
        c0 = j * 2
        _scatter_issue(0)
        _scatter_issue(1)
        _scatter_wait(0)

        @pl.when(c0 + 2 < NFULL)
        def _():
            _issue(c0 + 2, 0)

        _scatter_wait(1)

        @pl.when(c0 + 3 < NFULL)
        def _():
            _issue(c0 + 3, 1)

    toff = NFULL * 128
    pltpu.async_copy(g_hbm.at[sidx_v.at[pl.ds(toff, TAIL)]],
                     rows0_v.at[pl.ds(0, TAIL)], gsem0)
    pltpu.sync_copy(dst_hbm.at[pl.ds(base_e + toff, TAIL)], dtail_v)
    pltpu.make_async_copy(g_hbm.at[pl.ds(0, TAIL)],
                          rows0_v.at[pl.ds(0, TAIL)], gsem0).wait()
    pltpu.sync_copy(rows0_v.at[pl.ds(0, TAIL)], acc.at[dtail_v], add=True)

    plsc.subcore_barrier()
    pltpu.sync_copy(acc.at[pl.ds(row0, ROWS_PER_TILE)],
                    out_hbm.at[c, pl.ds(row0, ROWS_PER_TILE)])


_BLK = 2000


def _transform_body(dsrc_ref, feat_ref, w_ref, out_ref):
    deg = jnp.sum(dsrc_ref[...], axis=1, keepdims=True)
    norm = lax.rsqrt(jnp.maximum(deg, 1.0))
    x = feat_ref[...] * norm
    w = w_ref[...]
    cols = []
    for k in range(K):
        xk = x[:, k * IN:(k + 1) * IN]
        cols.append(lax.dot_general(
            xk, w[k], (((1,), (0,)), ((), ())),
            precision=lax.Precision.HIGHEST,
            preferred_element_type=jnp.float32))
    out_ref[...] = jnp.concatenate(cols, axis=1)


def _finish_body(parts_ref, ddst_ref, bias_ref, out_ref):
    ssum = parts_ref[0] + parts_ref[1]
    deg = jnp.sum(ddst_ref[...], axis=1, keepdims=True)
    norm = lax.rsqrt(jnp.maximum(deg, 1.0))
    out_ref[...] = ssum * norm + bias_ref[...]


def _tc_transform(dsrc, feat, weight):
    return pl.pallas_call(
        _transform_body,
        grid=(N // _BLK,),
        in_specs=[
            pl.BlockSpec((_BLK, NW), lambda i: (i, 0)),
            pl.BlockSpec((_BLK, F), lambda i: (i, 0)),
            pl.BlockSpec((K, IN, OUT), lambda i: (0, 0, 0)),
        ],
        out_specs=pl.BlockSpec((_BLK, F), lambda i: (i, 0)),
        out_shape=jax.ShapeDtypeStruct((N, F), jnp.float32),
    )(dsrc, feat, weight)


def _tc_finish(parts, ddst, bias2d):
    return pl.pallas_call(
        _finish_body,
        grid=(N // _BLK,),
        in_specs=[
            pl.BlockSpec((NC, _BLK, F), lambda i: (0, i, 0)),
            pl.BlockSpec((_BLK, NW), lambda i: (i, 0)),
            pl.BlockSpec((1, F), lambda i: (0, 0)),
        ],
        out_specs=pl.BlockSpec((_BLK, F), lambda i: (i, 0)),
        out_shape=jax.ShapeDtypeStruct((N, F), jnp.float32),
    )(parts, ddst, bias2d)


def kernel(feat, edge_index, weight, bias):
    src = edge_index[0]
    dst = edge_index[1]
    hist = _sc_degrees(src, dst)
    dsrc = hist[:, 0, :N].T
    ddst = hist[:, 1, :N].T
    g = _tc_transform(dsrc, feat, weight)
    parts = _sc_aggregate(g, src, dst)
    return _tc_finish(parts, ddst, bias.reshape(1, F))


# speedup vs baseline: 1.1849x; 1.1849x over previous
import functools

import jax
import jax.numpy as jnp
from jax import lax
from jax.experimental import pallas as pl
from jax.experimental.pallas import tpu as pltpu
from jax.experimental.pallas import tpu_sc as plsc

N = 10000
E = 320000
K = 4
IN = 32
OUT = 32
F = K * IN

NC = 2
NS = 16
NW = NC * NS

EPT = E // NW
CHUNK = 80
NCHUNK = EPT // CHUNK

HROWS = 640
ROWS_PER_TILE = N // NS

_MESH = plsc.VectorSubcoreMesh(core_axis_name="c", subcore_axis_name="s")


@functools.partial(
    pl.kernel,
    out_type=jax.ShapeDtypeStruct((NW, 2, HROWS * 16), jnp.float32),
    mesh=_MESH,
    scratch_types=[
        pltpu.VMEM((EPT,), jnp.int32),
        pltpu.VMEM((HROWS * 16,), jnp.float32),
        pltpu.VMEM((HROWS * 16,), jnp.float32),
    ],
    compiler_params=pltpu.CompilerParams(needs_layout_passes=False, use_tc_tiling_on_sc=False),
)
def _sc_degrees(src_hbm, dst_hbm, out_hbm, idx_v, hist_s, hist_d):
    c = lax.axis_index("c")
    s = lax.axis_index("s")
    wid = c * NS + s
    zero16 = jnp.zeros((16,), jnp.float32)
    ones16 = jnp.ones((16,), jnp.float32)

    @pl.loop(0, HROWS, unroll=8)
    def _zero(r):
        hist_s[pl.ds(r * 16, 16)] = zero16
        hist_d[pl.ds(r * 16, 16)] = zero16

    base_e = wid * EPT
    pltpu.sync_copy(src_hbm.at[pl.ds(base_e, EPT)], idx_v)

    @pl.loop(0, EPT // 16, unroll=8)
    def _hs(i):
        v = idx_v[pl.ds(i * 16, 16)]
        plsc.addupdate_scatter(hist_s, [v], ones16)

    pltpu.sync_copy(dst_hbm.at[pl.ds(base_e, EPT)], idx_v)

    @pl.loop(0, EPT // 16, unroll=8)
    def _hd(i):
        v = idx_v[pl.ds(i * 16, 16)]
        plsc.addupdate_scatter(hist_d, [v], ones16)

    pltpu.sync_copy(hist_s, out_hbm.at[wid, 0])
    pltpu.sync_copy(hist_d, out_hbm.at[wid, 1])


NFULL = EPT // 128
TAIL = EPT - NFULL * 128


@functools.partial(
    pl.kernel,
    out_type=jax.ShapeDtypeStruct((NC, N, F), jnp.float32),
    mesh=_MESH,
    scratch_types=[
        pltpu.VMEM((EPT,), jnp.int32),
        pltpu.VMEM((128,), jnp.int32),
        pltpu.VMEM((128,), jnp.int32),
        pltpu.VMEM((TAIL,), jnp.int32),
        pltpu.VMEM((128, F), jnp.float32),
        pltpu.VMEM((128, F), jnp.float32),
        pltpu.VMEM((25, F), jnp.float32),
        pltpu.VMEM_SHARED((N, F), jnp.float32),
        pltpu.SemaphoreType.DMA,
        pltpu.SemaphoreType.DMA,
        pltpu.SemaphoreType.DMA,
        pltpu.SemaphoreType.DMA,
    ],
    compiler_params=pltpu.CompilerParams(needs_layout_passes=False, use_tc_tiling_on_sc=False),
)
def _sc_aggregate(g_hbm, src_hbm, dst_hbm, out_hbm,
                  sidx_v, dwin0_v, dwin1_v, dtail_v, rows0_v, rows1_v,
                  zbuf_v, acc, gsem0, gsem1, dsem0, dsem1):
    c = lax.axis_index("c")
    s = lax.axis_index("s")
    wid = c * NS + s
    zero16 = jnp.zeros((16,), jnp.float32)

    @pl.loop(0, 25)
    def _zr(r):
        @pl.loop(0, F // 16)
        def _zc(q):
            zbuf_v[r, pl.ds(q * 16, 16)] = zero16

    row0 = s * ROWS_PER_TILE

    @pl.loop(0, ROWS_PER_TILE // 25)
    def _za(b):
        pltpu.sync_copy(zbuf_v, acc.at[pl.ds(row0 + b * 25, 25)])

    base_e = wid * EPT
    pltpu.sync_copy(src_hbm.at[pl.ds(base_e, EPT)], sidx_v)

    plsc.subcore_barrier()

    rows = (rows0_v, rows1_v)
    dwin = (dwin0_v, dwin1_v)
    gsems = (gsem0, gsem1)
    dsems = (dsem0, dsem1)

    def _issue(cc, b):
        off = pl.multiple_of(cc * 128, 8)
        pltpu.async_copy(g_hbm.at[sidx_v.at[pl.ds(off, 128)]], rows[b],
                         gsems[b])
        pltpu.async_copy(dst_hbm.at[pl.ds(base_e + off, 128)], dwin[b],
                         dsems[b])

    def _drain_scatter(b):
        pltpu.make_async_copy(g_hbm.at[pl.ds(0, 128)], rows[b],
                              gsems[b]).wait()
        pltpu.make_async_copy(dst_hbm.at[pl.ds(0, 128)], dwin[b],
                              dsems[b]).wait()
        pltpu.sync_copy(rows[b], acc.at[dwin[b]], add=True)

    _issue(0, 0)

    @pl.loop(0, NFULL // 2)
    def _chunks(j):
        c0 = j * 2
        _issue(c0 + 1, 1)
        _drain_scatter(0)

        @pl.when(c0 + 2 < NFULL)
        def _():
            _issue(c0 + 2, 0)

        _drain_scatter(1)

    toff = NFULL * 128
    pltpu.async_copy(g_hbm.at[sidx_v.at[pl.ds(toff, TAIL)]],
                     rows0_v.at[pl.ds(0, TAIL)], gsem0)
    pltpu.sync_copy(dst_hbm.at[pl.ds(base_e + toff, TAIL)], dtail_v)
    pltpu.make_async_copy(g_hbm.at[pl.ds(0, TAIL)],
                          rows0_v.at[pl.ds(0, TAIL)], gsem0).wait()
    pltpu.sync_copy(rows0_v.at[pl.ds(0, TAIL)], acc.at[dtail_v], add=True)

    plsc.subcore_barrier()
    pltpu.sync_copy(acc.at[pl.ds(row0, ROWS_PER_TILE)],
                    out_hbm.at[c, pl.ds(row0, ROWS_PER_TILE)])


_BLK = 2000


def _transform_body(dsrc_ref, feat_ref, w_ref, out_ref):
    deg = jnp.sum(dsrc_ref[...], axis=1, keepdims=True)
    norm = lax.rsqrt(jnp.maximum(deg, 1.0))
    x = feat_ref[...] * norm
    w = w_ref[...]
    cols = []
    for k in range(K):
        xk = x[:, k * IN:(k + 1) * IN]
        cols.append(lax.dot_general(
            xk, w[k], (((1,), (0,)), ((), ())),
            precision=lax.Precision.HIGHEST,
            preferred_element_type=jnp.float32))
    out_ref[...] = jnp.concatenate(cols, axis=1)


def _finish_body(parts_ref, ddst_ref, bias_ref, out_ref):
    ssum = parts_ref[0] + parts_ref[1]
    deg = jnp.sum(ddst_ref[...], axis=1, keepdims=True)
    norm = lax.rsqrt(jnp.maximum(deg, 1.0))
    out_ref[...] = ssum * norm + bias_ref[...]


def _tc_transform(dsrc, feat, weight):
    return pl.pallas_call(
        _transform_body,
        grid=(N // _BLK,),
        in_specs=[
            pl.BlockSpec((_BLK, NW), lambda i: (i, 0)),
            pl.BlockSpec((_BLK, F), lambda i: (i, 0)),
            pl.BlockSpec((K, IN, OUT), lambda i: (0, 0, 0)),
        ],
        out_specs=pl.BlockSpec((_BLK, F), lambda i: (i, 0)),
        out_shape=jax.ShapeDtypeStruct((N, F), jnp.float32),
    )(dsrc, feat, weight)


def _tc_finish(parts, ddst, bias2d):
    return pl.pallas_call(
        _finish_body,
        grid=(N // _BLK,),
        in_specs=[
            pl.BlockSpec((NC, _BLK, F), lambda i: (0, i, 0)),
            pl.BlockSpec((_BLK, NW), lambda i: (i, 0)),
            pl.BlockSpec((1, F), lambda i: (0, 0)),
        ],
        out_specs=pl.BlockSpec((_BLK, F), lambda i: (i, 0)),
        out_shape=jax.ShapeDtypeStruct((N, F), jnp.float32),
    )(parts, ddst, bias2d)


def kernel(feat, edge_index, weight, bias):
    src = edge_index[0]
    dst = edge_index[1]
    hist = _sc_degrees(src, dst)
    dsrc = hist[:, 0, :N].T
    ddst = hist[:, 1, :N].T
    g = _tc_transform(dsrc, feat, weight)
    parts = _sc_aggregate(g, src, dst)
    return _tc_finish(parts, ddst, bias.reshape(1, F))


# speedup vs baseline: 1.2470x; 1.0524x over previous
import functools

import jax
import jax.numpy as jnp
from jax import lax
from jax.experimental import pallas as pl
from jax.experimental.pallas import tpu as pltpu
from jax.experimental.pallas import tpu_sc as plsc

N = 10000
E = 320000
K = 4
IN = 32
OUT = 32
F = K * IN

NC = 2
NS = 16
NW = NC * NS

EPT = E // NW
CHUNK = 80
NCHUNK = EPT // CHUNK

HROWS = 640
ROWS_PER_TILE = N // NS

_MESH = plsc.VectorSubcoreMesh(core_axis_name="c", subcore_axis_name="s")


@functools.partial(
    pl.kernel,
    out_type=jax.ShapeDtypeStruct((NW, 2, HROWS * 16), jnp.float32),
    mesh=_MESH,
    scratch_types=[
        pltpu.VMEM((EPT,), jnp.int32),
        pltpu.VMEM((HROWS * 16,), jnp.float32),
        pltpu.VMEM((HROWS * 16,), jnp.float32),
    ],
    compiler_params=pltpu.CompilerParams(needs_layout_passes=False, use_tc_tiling_on_sc=False),
)
def _sc_degrees(edge_hbm, out_hbm, idx_v, hist_s, hist_d):
    c = lax.axis_index("c")
    s = lax.axis_index("s")
    wid = c * NS + s
    zero16 = jnp.zeros((16,), jnp.float32)
    ones16 = jnp.ones((16,), jnp.float32)

    @pl.loop(0, HROWS, unroll=8)
    def _zero(r):
        hist_s[pl.ds(r * 16, 16)] = zero16
        hist_d[pl.ds(r * 16, 16)] = zero16

    base_e = wid * EPT
    pltpu.sync_copy(edge_hbm.at[0, pl.ds(base_e, EPT)], idx_v)

    @pl.loop(0, EPT // 16, unroll=8)
    def _hs(i):
        v = idx_v[pl.ds(i * 16, 16)]
        plsc.addupdate_scatter(hist_s, [v], ones16)

    pltpu.sync_copy(edge_hbm.at[1, pl.ds(base_e, EPT)], idx_v)

    @pl.loop(0, EPT // 16, unroll=8)
    def _hd(i):
        v = idx_v[pl.ds(i * 16, 16)]
        plsc.addupdate_scatter(hist_d, [v], ones16)

    pltpu.sync_copy(hist_s, out_hbm.at[wid, 0])
    pltpu.sync_copy(hist_d, out_hbm.at[wid, 1])


NFULL = EPT // 128
TAIL = EPT - NFULL * 128


@functools.partial(
    pl.kernel,
    out_type=jax.ShapeDtypeStruct((NC, N, F), jnp.float32),
    mesh=_MESH,
    scratch_types=[
        pltpu.VMEM((EPT,), jnp.int32),
        pltpu.VMEM((128,), jnp.int32),
        pltpu.VMEM((128,), jnp.int32),
        pltpu.VMEM((TAIL,), jnp.int32),
        pltpu.VMEM((128, F), jnp.float32),
        pltpu.VMEM((128, F), jnp.float32),
        pltpu.VMEM((25, F), jnp.float32),
        pltpu.VMEM_SHARED((N, F), jnp.float32),
        pltpu.SemaphoreType.DMA,
        pltpu.SemaphoreType.DMA,
        pltpu.SemaphoreType.DMA,
        pltpu.SemaphoreType.DMA,
    ],
    compiler_params=pltpu.CompilerParams(needs_layout_passes=False, use_tc_tiling_on_sc=False),
)
def _sc_aggregate(g_hbm, edge_hbm, out_hbm,
                  sidx_v, dwin0_v, dwin1_v, dtail_v, rows0_v, rows1_v,
                  zbuf_v, acc, gsem0, gsem1, dsem0, dsem1):
    c = lax.axis_index("c")
    s = lax.axis_index("s")
    wid = c * NS + s
    zero16 = jnp.zeros((16,), jnp.float32)

    @pl.loop(0, 25)
    def _zr(r):
        @pl.loop(0, F // 16)
        def _zc(q):
            zbuf_v[r, pl.ds(q * 16, 16)] = zero16

    row0 = s * ROWS_PER_TILE

    @pl.loop(0, ROWS_PER_TILE // 25)
    def _za(b):
        pltpu.sync_copy(zbuf_v, acc.at[pl.ds(row0 + b * 25, 25)])

    base_e = wid * EPT
    pltpu.sync_copy(edge_hbm.at[0, pl.ds(base_e, EPT)], sidx_v)

    plsc.subcore_barrier()

    rows = (rows0_v, rows1_v)
    dwin = (dwin0_v, dwin1_v)
    gsems = (gsem0, gsem1)
    dsems = (dsem0, dsem1)

    def _issue(cc, b):
        off = pl.multiple_of(cc * 128, 8)
        pltpu.async_copy(g_hbm.at[sidx_v.at[pl.ds(off, 128)]], rows[b],
                         gsems[b])
        pltpu.async_copy(edge_hbm.at[1, pl.ds(base_e + off, 128)], dwin[b],
                         dsems[b])

    def _drain_scatter(b):
        pltpu.make_async_copy(g_hbm.at[pl.ds(0, 128)], rows[b],
                              gsems[b]).wait()
        pltpu.make_async_copy(edge_hbm.at[1, pl.ds(0, 128)], dwin[b],
                              dsems[b]).wait()
        pltpu.sync_copy(rows[b], acc.at[dwin[b]], add=True)

    _issue(0, 0)

    @pl.loop(0, NFULL // 2)
    def _chunks(j):
        c0 = j * 2
        _issue(c0 + 1, 1)
        _drain_scatter(0)

        @pl.when(c0 + 2 < NFULL)
        def _():
            _issue(c0 + 2, 0)

        _drain_scatter(1)

    toff = NFULL * 128
    pltpu.async_copy(g_hbm.at[sidx_v.at[pl.ds(toff, TAIL)]],
                     rows0_v.at[pl.ds(0, TAIL)], gsem0)
    pltpu.sync_copy(edge_hbm.at[1, pl.ds(base_e + toff, TAIL)], dtail_v)
    pltpu.make_async_copy(g_hbm.at[pl.ds(0, TAIL)],
                          rows0_v.at[pl.ds(0, TAIL)], gsem0).wait()
    pltpu.sync_copy(rows0_v.at[pl.ds(0, TAIL)], acc.at[dtail_v], add=True)

    plsc.subcore_barrier()
    pltpu.sync_copy(acc.at[pl.ds(row0, ROWS_PER_TILE)],
                    out_hbm.at[c, pl.ds(row0, ROWS_PER_TILE)])


_BLK = 2000


def _transform_body(dsrc_ref, feat_ref, w_ref, out_ref):
    deg = jnp.sum(dsrc_ref[...], axis=1, keepdims=True)
    norm = lax.rsqrt(jnp.maximum(deg, 1.0))
    x = feat_ref[...] * norm
    w = w_ref[...]
    cols = []
    for k in range(K):
        xk = x[:, k * IN:(k + 1) * IN]
        cols.append(lax.dot_general(
            xk, w[k], (((1,), (0,)), ((), ())),
            precision=lax.Precision.HIGHEST,
            preferred_element_type=jnp.float32))
    out_ref[...] = jnp.concatenate(cols, axis=1)


def _finish_body(parts_ref, ddst_ref, bias_ref, out_ref):
    ssum = parts_ref[0] + parts_ref[1]
    deg = jnp.sum(ddst_ref[...], axis=1, keepdims=True)
    norm = lax.rsqrt(jnp.maximum(deg, 1.0))
    out_ref[...] = ssum * norm + bias_ref[...]


def _tc_transform(dsrc, feat, weight):
    return pl.pallas_call(
        _transform_body,
        grid=(N // _BLK,),
        in_specs=[
            pl.BlockSpec((_BLK, NW), lambda i: (i, 0)),
            pl.BlockSpec((_BLK, F), lambda i: (i, 0)),
            pl.BlockSpec((K, IN, OUT), lambda i: (0, 0, 0)),
        ],
        out_specs=pl.BlockSpec((_BLK, F), lambda i: (i, 0)),
        out_shape=jax.ShapeDtypeStruct((N, F), jnp.float32),
    )(dsrc, feat, weight)


def _tc_finish(parts, ddst, bias2d):
    return pl.pallas_call(
        _finish_body,
        grid=(N // _BLK,),
        in_specs=[
            pl.BlockSpec((NC, _BLK, F), lambda i: (0, i, 0)),
            pl.BlockSpec((_BLK, NW), lambda i: (i, 0)),
            pl.BlockSpec((1, F), lambda i: (0, 0)),
        ],
        out_specs=pl.BlockSpec((_BLK, F), lambda i: (i, 0)),
        out_shape=jax.ShapeDtypeStruct((N, F), jnp.float32),
    )(parts, ddst, bias2d)


def kernel(feat, edge_index, weight, bias):
    hist = _sc_degrees(edge_index)
    dsrc = hist[:, 0, :N].T
    ddst = hist[:, 1, :N].T
    g = _tc_transform(dsrc, feat, weight)
    parts = _sc_aggregate(g, edge_index)
    return _tc_finish(parts, ddst, bias.reshape(1, F))


# speedup vs baseline: 1.2534x; 1.0052x over previous
import functools

import jax
import jax.numpy as jnp
from jax import lax
from jax.experimental import pallas as pl
from jax.experimental.pallas import tpu as pltpu
from jax.experimental.pallas import tpu_sc as plsc

N = 10000
E = 320000
K = 4
IN = 32
OUT = 32
F = K * IN

NC = 2
NS = 16
NW = NC * NS

EPT = E // NW
CHUNK = 80
NCHUNK = EPT // CHUNK

HROWS = 640
ROWS_PER_TILE = N // NS

_MESH = plsc.VectorSubcoreMesh(core_axis_name="c", subcore_axis_name="s")


@functools.partial(
    pl.kernel,
    out_type=jax.ShapeDtypeStruct((NW, 2, HROWS * 16), jnp.float32),
    mesh=_MESH,
    scratch_types=[
        pltpu.VMEM((EPT,), jnp.int32),
        pltpu.VMEM((HROWS * 16,), jnp.float32),
        pltpu.VMEM((HROWS * 16,), jnp.float32),
    ],
    compiler_params=pltpu.CompilerParams(needs_layout_passes=False, use_tc_tiling_on_sc=False),
)
def _sc_degrees(edge_hbm, out_hbm, idx_v, hist_s, hist_d):
    c = lax.axis_index("c")
    s = lax.axis_index("s")
    wid = c * NS + s
    zero16 = jnp.zeros((16,), jnp.float32)
    ones16 = jnp.ones((16,), jnp.float32)

    @pl.loop(0, HROWS, unroll=8)
    def _zero(r):
        hist_s[pl.ds(r * 16, 16)] = zero16
        hist_d[pl.ds(r * 16, 16)] = zero16

    base_e = wid * EPT
    pltpu.sync_copy(edge_hbm.at[0, pl.ds(base_e, EPT)], idx_v)

    @pl.loop(0, EPT // 16, unroll=8)
    def _hs(i):
        v = idx_v[pl.ds(i * 16, 16)]
        plsc.addupdate_scatter(hist_s, [v], ones16)

    pltpu.sync_copy(edge_hbm.at[1, pl.ds(base_e, EPT)], idx_v)

    @pl.loop(0, EPT // 16, unroll=8)
    def _hd(i):
        v = idx_v[pl.ds(i * 16, 16)]
        plsc.addupdate_scatter(hist_d, [v], ones16)

    pltpu.sync_copy(hist_s, out_hbm.at[wid, 0])
    pltpu.sync_copy(hist_d, out_hbm.at[wid, 1])


NFULL = EPT // 128
TAIL = EPT - NFULL * 128


@functools.partial(
    pl.kernel,
    out_type=jax.ShapeDtypeStruct((NC, N, F), jnp.float32),
    mesh=_MESH,
    scratch_types=[
        pltpu.VMEM((EPT,), jnp.int32),
        pltpu.VMEM((128,), jnp.int32),
        pltpu.VMEM((128,), jnp.int32),
        pltpu.VMEM((TAIL,), jnp.int32),
        pltpu.VMEM((128, F), jnp.float32),
        pltpu.VMEM((128, F), jnp.float32),
        pltpu.VMEM((25, F), jnp.float32),
        pltpu.VMEM_SHARED((N, F), jnp.float32),
        pltpu.SemaphoreType.DMA,
        pltpu.SemaphoreType.DMA,
        pltpu.SemaphoreType.DMA,
        pltpu.SemaphoreType.DMA,
    ],
    compiler_params=pltpu.CompilerParams(needs_layout_passes=False, use_tc_tiling_on_sc=False),
)
def _sc_aggregate(g_hbm, edge_hbm, out_hbm,
                  sidx_v, dwin0_v, dwin1_v, dtail_v, rows0_v, rows1_v,
                  zbuf_v, acc, gsem0, gsem1, dsem0, dsem1):
    c = lax.axis_index("c")
    s = lax.axis_index("s")
    wid = c * NS + s
    zero16 = jnp.zeros((16,), jnp.float32)

    base_e = wid * EPT
    pltpu.async_copy(edge_hbm.at[0, pl.ds(base_e, EPT)], sidx_v, gsem0)

    @pl.loop(0, 25)
    def _zr(r):
        @pl.loop(0, F // 16)
        def _zc(q):
            zbuf_v[r, pl.ds(q * 16, 16)] = zero16

    row0 = s * ROWS_PER_TILE

    @pl.loop(0, ROWS_PER_TILE // 25)
    def _za(b):
        pltpu.sync_copy(zbuf_v, acc.at[pl.ds(row0 + b * 25, 25)])

    pltpu.make_async_copy(edge_hbm.at[0, pl.ds(base_e, EPT)], sidx_v,
                          gsem0).wait()

    plsc.subcore_barrier()

    rows = (rows0_v, rows1_v)
    dwin = (dwin0_v, dwin1_v)
    gsems = (gsem0, gsem1)
    dsems = (dsem0, dsem1)

    def _issue(cc, b):
        off = pl.multiple_of(cc * 128, 8)
        pltpu.async_copy(g_hbm.at[sidx_v.at[pl.ds(off, 128)]], rows[b],
                         gsems[b])
        pltpu.async_copy(edge_hbm.at[1, pl.ds(base_e + off, 128)], dwin[b],
                         dsems[b])

    def _drain_scatter(b):
        pltpu.make_async_copy(g_hbm.at[pl.ds(0, 128)], rows[b],
                              gsems[b]).wait()
        pltpu.make_async_copy(edge_hbm.at[1, pl.ds(0, 128)], dwin[b],
                              dsems[b]).wait()
        pltpu.sync_copy(rows[b], acc.at[dwin[b]], add=True)

    _issue(0, 0)

    @pl.loop(0, NFULL // 2)
    def _chunks(j):
        c0 = j * 2
        _issue(c0 + 1, 1)
        _drain_scatter(0)

        @pl.when(c0 + 2 < NFULL)
        def _():
            _issue(c0 + 2, 0)

        _drain_scatter(1)

    toff = NFULL * 128
    pltpu.async_copy(g_hbm.at[sidx_v.at[pl.ds(toff, TAIL)]],
                     rows0_v.at[pl.ds(0, TAIL)], gsem0)
    pltpu.sync_copy(edge_hbm.at[1, pl.ds(base_e + toff, TAIL)], dtail_v)
    pltpu.make_async_copy(g_hbm.at[pl.ds(0, TAIL)],
                          rows0_v.at[pl.ds(0, TAIL)], gsem0).wait()
    pltpu.sync_copy(rows0_v.at[pl.ds(0, TAIL)], acc.at[dtail_v], add=True)

    plsc.subcore_barrier()
    pltpu.sync_copy(acc.at[pl.ds(row0, ROWS_PER_TILE)],
                    out_hbm.at[c, pl.ds(row0, ROWS_PER_TILE)])


_BLK = 2000


def _transform_body(dsrc_ref, feat_ref, w_ref, out_ref):
    deg = jnp.sum(dsrc_ref[...], axis=1, keepdims=True)
    norm = lax.rsqrt(jnp.maximum(deg, 1.0))
    x = feat_ref[...] * norm
    w = w_ref[...]
    cols = []
    for k in range(K):
        xk = x[:, k * IN:(k + 1) * IN]
        cols.append(lax.dot_general(
            xk, w[k], (((1,), (0,)), ((), ())),
            precision=lax.Precision.HIGHEST,
            preferred_element_type=jnp.float32))
    out_ref[...] = jnp.concatenate(cols, axis=1)


def _finish_body(parts_ref, ddst_ref, bias_ref, out_ref):
    ssum = parts_ref[0] + parts_ref[1]
    deg = jnp.sum(ddst_ref[...], axis=1, keepdims=True)
    norm = lax.rsqrt(jnp.maximum(deg, 1.0))
    out_ref[...] = ssum * norm + bias_ref[...]


def _tc_transform(dsrc, feat, weight):
    return pl.pallas_call(
        _transform_body,
        grid=(N // _BLK,),
        in_specs=[
            pl.BlockSpec((_BLK, NW), lambda i: (i, 0)),
            pl.BlockSpec((_BLK, F), lambda i: (i, 0)),
            pl.BlockSpec((K, IN, OUT), lambda i: (0, 0, 0)),
        ],
        out_specs=pl.BlockSpec((_BLK, F), lambda i: (i, 0)),
        out_shape=jax.ShapeDtypeStruct((N, F), jnp.float32),
    )(dsrc, feat, weight)


def _tc_finish(parts, ddst, bias2d):
    return pl.pallas_call(
        _finish_body,
        grid=(N // _BLK,),
        in_specs=[
            pl.BlockSpec((NC, _BLK, F), lambda i: (0, i, 0)),
            pl.BlockSpec((_BLK, NW), lambda i: (i, 0)),
            pl.BlockSpec((1, F), lambda i: (0, 0)),
        ],
        out_specs=pl.BlockSpec((_BLK, F), lambda i: (i, 0)),
        out_shape=jax.ShapeDtypeStruct((N, F), jnp.float32),
    )(parts, ddst, bias2d)


def kernel(feat, edge_index, weight, bias):
    hist = _sc_degrees(edge_index)
    dsrc = hist[:, 0, :N].T
    ddst = hist[:, 1, :N].T
    g = _tc_transform(dsrc, feat, weight)
    parts = _sc_aggregate(g, edge_index)
    return _tc_finish(parts, ddst, bias.reshape(1, F))


# speedup vs baseline: 1.2537x; 1.0002x over previous
import functools

import jax
import jax.numpy as jnp
from jax import lax
from jax.experimental import pallas as pl
from jax.experimental.pallas import tpu as pltpu
from jax.experimental.pallas import tpu_sc as plsc

N = 10000
E = 320000
K = 4
IN = 32
OUT = 32
F = K * IN

NC = 2
NS = 16
NW = NC * NS

EPT = E // NW

HROWS = 640
ROWS_PER_TILE = N // NS

_MESH = plsc.VectorSubcoreMesh(core_axis_name="c", subcore_axis_name="s")


@functools.partial(
    pl.kernel,
    out_type=jax.ShapeDtypeStruct((NW, 2, HROWS * 16), jnp.float32),
    mesh=_MESH,
    scratch_types=[
        pltpu.VMEM((EPT,), jnp.int32),
        pltpu.VMEM((HROWS * 16,), jnp.float32),
        pltpu.VMEM((HROWS * 16,), jnp.float32),
    ],
    compiler_params=pltpu.CompilerParams(needs_layout_passes=False, use_tc_tiling_on_sc=False),
)
def _sc_degrees(edge_hbm, out_hbm, idx_v, hist_s, hist_d):
    c = lax.axis_index("c")
    s = lax.axis_index("s")
    wid = c * NS + s
    zero16 = jnp.zeros((16,), jnp.float32)
    ones16 = jnp.ones((16,), jnp.float32)

    @pl.loop(0, HROWS, unroll=8)
    def _zero(r):
        hist_s[pl.ds(r * 16, 16)] = zero16
        hist_d[pl.ds(r * 16, 16)] = zero16

    base_e = wid * EPT
    pltpu.sync_copy(edge_hbm.at[0, pl.ds(base_e, EPT)], idx_v)

    @pl.loop(0, EPT // 16, unroll=8)
    def _hs(i):
        v = idx_v[pl.ds(i * 16, 16)]
        plsc.addupdate_scatter(hist_s, [v], ones16)

    pltpu.sync_copy(edge_hbm.at[1, pl.ds(base_e, EPT)], idx_v)

    @pl.loop(0, EPT // 16, unroll=8)
    def _hd(i):
        v = idx_v[pl.ds(i * 16, 16)]
        plsc.addupdate_scatter(hist_d, [v], ones16)

    pltpu.sync_copy(hist_s, out_hbm.at[wid, 0])
    pltpu.sync_copy(hist_d, out_hbm.at[wid, 1])


NFULL = EPT // 128
TAIL = EPT - NFULL * 128


@functools.partial(
    pl.kernel,
    out_type=jax.ShapeDtypeStruct((NC, N, F), jnp.float32),
    mesh=_MESH,
    scratch_types=[
        pltpu.VMEM((EPT,), jnp.int32),
        pltpu.VMEM((128,), jnp.int32),
        pltpu.VMEM((128,), jnp.int32),
        pltpu.VMEM((TAIL,), jnp.int32),
        pltpu.VMEM((128, F), jnp.float32),
        pltpu.VMEM((128, F), jnp.float32),
        pltpu.VMEM((25, F), jnp.float32),
        pltpu.VMEM_SHARED((N, F), jnp.float32),
        pltpu.SemaphoreType.DMA,
        pltpu.SemaphoreType.DMA,
        pltpu.SemaphoreType.DMA,
        pltpu.SemaphoreType.DMA,
    ],
    compiler_params=pltpu.CompilerParams(needs_layout_passes=False, use_tc_tiling_on_sc=False),
)
def _sc_aggregate(g_hbm, edge_hbm, out_hbm,
                  sidx_v, dwin0_v, dwin1_v, dtail_v, rows0_v, rows1_v,
                  zbuf_v, acc, gsem0, gsem1, dsem0, dsem1):
    c = lax.axis_index("c")
    s = lax.axis_index("s")
    wid = c * NS + s
    zero16 = jnp.zeros((16,), jnp.float32)

    base_e = wid * EPT
    pltpu.async_copy(edge_hbm.at[0, pl.ds(base_e, EPT)], sidx_v, gsem0)

    @pl.loop(0, 25)
    def _zr(r):
        @pl.loop(0, F // 16)
        def _zc(q):
            zbuf_v[r, pl.ds(q * 16, 16)] = zero16

    row0 = s * ROWS_PER_TILE

    @pl.loop(0, ROWS_PER_TILE // 25)
    def _za(b):
        pltpu.sync_copy(zbuf_v, acc.at[pl.ds(row0 + b * 25, 25)])

    pltpu.make_async_copy(edge_hbm.at[0, pl.ds(base_e, EPT)], sidx_v,
                          gsem0).wait()

    plsc.subcore_barrier()

    rows = (rows0_v, rows1_v)
    dwin = (dwin0_v, dwin1_v)
    gsems = (gsem0, gsem1)
    dsems = (dsem0, dsem1)

    def _issue(cc, b):
        off = pl.multiple_of(cc * 128, 8)
        pltpu.async_copy(g_hbm.at[sidx_v.at[pl.ds(off, 128)]], rows[b],
                         gsems[b])
        pltpu.async_copy(edge_hbm.at[1, pl.ds(base_e + off, 128)], dwin[b],
                         dsems[b])

    def _drain_scatter(b):
        pltpu.make_async_copy(g_hbm.at[pl.ds(0, 128)], rows[b],
                              gsems[b]).wait()
        pltpu.make_async_copy(edge_hbm.at[1, pl.ds(0, 128)], dwin[b],
                              dsems[b]).wait()
        pltpu.sync_copy(rows[b], acc.at[dwin[b]], add=True)

    _issue(0, 0)

    @pl.loop(0, NFULL // 2)
    def _chunks(j):
        c0 = j * 2
        _issue(c0 + 1, 1)
        _drain_scatter(0)

        @pl.when(c0 + 2 < NFULL)
        def _():
            _issue(c0 + 2, 0)

        _drain_scatter(1)

    toff = NFULL * 128
    pltpu.async_copy(g_hbm.at[sidx_v.at[pl.ds(toff, TAIL)]],
                     rows0_v.at[pl.ds(0, TAIL)], gsem0)
    pltpu.sync_copy(edge_hbm.at[1, pl.ds(base_e + toff, TAIL)], dtail_v)
    pltpu.make_async_copy(g_hbm.at[pl.ds(0, TAIL)],
                          rows0_v.at[pl.ds(0, TAIL)], gsem0).wait()
    pltpu.sync_copy(rows0_v.at[pl.ds(0, TAIL)], acc.at[dtail_v], add=True)

    plsc.subcore_barrier()
    pltpu.sync_copy(acc.at[pl.ds(row0, ROWS_PER_TILE)],
                    out_hbm.at[c, pl.ds(row0, ROWS_PER_TILE)])


_BLK = 2000


def _transform_body(dsrc_ref, feat_ref, w_ref, out_ref):
    deg = jnp.sum(dsrc_ref[...], axis=1, keepdims=True)
    norm = lax.rsqrt(jnp.maximum(deg, 1.0))
    x = feat_ref[...] * norm
    w = w_ref[...]
    cols = []
    for k in range(K):
        xk = x[:, k * IN:(k + 1) * IN]
        cols.append(lax.dot_general(
            xk, w[k], (((1,), (0,)), ((), ())),
            precision=lax.Precision.HIGHEST,
            preferred_element_type=jnp.float32))
    out_ref[...] = jnp.concatenate(cols, axis=1)


def _finish_body(parts_ref, ddst_ref, bias_ref, out_ref):
    ssum = parts_ref[0] + parts_ref[1]
    deg = jnp.sum(ddst_ref[...], axis=1, keepdims=True)
    norm = lax.rsqrt(jnp.maximum(deg, 1.0))
    out_ref[...] = ssum * norm + bias_ref[...]


def _tc_transform(dsrc, feat, weight):
    return pl.pallas_call(
        _transform_body,
        grid=(N // _BLK,),
        in_specs=[
            pl.BlockSpec((_BLK, NW), lambda i: (i, 0)),
            pl.BlockSpec((_BLK, F), lambda i: (i, 0)),
            pl.BlockSpec((K, IN, OUT), lambda i: (0, 0, 0)),
        ],
        out_specs=pl.BlockSpec((_BLK, F), lambda i: (i, 0)),
        out_shape=jax.ShapeDtypeStruct((N, F), jnp.float32),
    )(dsrc, feat, weight)


def _tc_finish(parts, ddst, bias2d):
    return pl.pallas_call(
        _finish_body,
        grid=(N // _BLK,),
        in_specs=[
            pl.BlockSpec((NC, _BLK, F), lambda i: (0, i, 0)),
            pl.BlockSpec((_BLK, NW), lambda i: (i, 0)),
            pl.BlockSpec((1, F), lambda i: (0, 0)),
        ],
        out_specs=pl.BlockSpec((_BLK, F), lambda i: (i, 0)),
        out_shape=jax.ShapeDtypeStruct((N, F), jnp.float32),
    )(parts, ddst, bias2d)


def kernel(feat, edge_index, weight, bias):
    hist = _sc_degrees(edge_index)
    dsrc = hist[:, 0, :N].T
    ddst = hist[:, 1, :N].T
    g = _tc_transform(dsrc, feat, weight)
    parts = _sc_aggregate(g, edge_index)
    return _tc_finish(parts, ddst, bias.reshape(1, F))
